# baseline (speedup 1.0000x reference)
import functools

import jax
import jax.numpy as jnp
from jax import lax
from jax.experimental import pallas as pl
from jax.experimental.pallas import tpu as pltpu
from jax.experimental.pallas import tpu_sc as plsc

_M = 7
_TAU = 3


def kernel(x):
    B, S, C = x.shape
    nwin = _M + 1
    halo = _M * _TAU
    valid_t = S - halo
    nsc = 32
    ttile = 64
    tch = 24
    glen = 48
    nrow = B * nwin * (C // 128)
    ngr = nrow // 8
    last_t0 = ((valid_t - ttile - 1) // 8 + 1) * 8
    tail_l2 = valid_t - last_t0 - 2 * tch

    mesh = plsc.VectorSubcoreMesh(core_axis_name="c", subcore_axis_name="s")

    @functools.partial(
        pl.kernel,
        mesh=mesh,
        out_type=jax.ShapeDtypeStruct((valid_t, nrow, 128), jnp.float32),
        scratch_types=[
            pltpu.VMEM((B, glen, C), jnp.float32),
            pltpu.VMEM((2, tch, 8, 128), jnp.float32),
            pltpu.SemaphoreType.DMA,
            pltpu.SemaphoreType.DMA,
        ],
    )
    def run(x_hbm, out_hbm, buf_in, buf_st, gsem, ssem):
        cc_ = lax.axis_index("c")
        ss_ = lax.axis_index("s")
        wid = ss_ * 2 + cc_
        t0 = jnp.where(wid < nsc - 1, wid * ttile, last_t0)

        def wait_scatter(rows):
            pltpu.make_async_copy(
                buf_st.at[0, pl.ds(0, rows), :, :],
                out_hbm.at[pl.ds(0, rows), pl.ds(0, 8), :],
                ssem,
            ).wait()

        def emit_groups(t0c, lc, first_chunk, prev_rows):
            for g in range(ngr):
                slot = g % 2
                if first_chunk is None:
                    if g < 2:
                        wait_scatter(prev_rows)
                    else:
                        wait_scatter(lc)
                else:
                    if g < 2:
                        @pl.when(jnp.logical_not(first_chunk))
                        def _w():
                            wait_scatter(prev_rows)
                    else:
                        wait_scatter(lc)

                @plsc.parallel_loop(0, lc)
                def rot_body(toff):
                    for half in range(2):
                        vals = []
                        for r8 in range(8):
                            row = 8 * g + r8
                            bb, col = row % 2, row // 2
                            win, c6 = col // (C // 128), col % (C // 128)
                            for w in range(4):
                                vals.append(
                                    buf_in[
                                        bb,
                                        _TAU * win + toff,
                                        pl.ds(128 * c6 + 64 * half + 16 * w, 16),
                                    ]
                                )
                        idx = 0
                        for r8 in range(8):
                            for w in range(4):
                                buf_st[
                                    slot,
                                    toff,
                                    r8,
                                    pl.ds(64 * half + 16 * w, 16),
                                ] = vals[idx]
                                idx += 1

                pltpu.async_copy(
                    buf_st.at[slot, pl.ds(0, lc), :, :],
                    out_hbm.at[pl.ds(t0c, lc), pl.ds(8 * g, 8), :],
                    ssem,
                )

        def wait_gather(rows):
            pltpu.make_async_copy(
                x_hbm.at[:, pl.ds(0, rows), :],
                buf_in.at[:, pl.ds(0, rows), :],
                gsem,
            ).wait()

        pltpu.async_copy(
            x_hbm.at[:, pl.ds(pl.multiple_of(t0, 8), glen), :], buf_in, gsem
        ).wait()

        def chunk_body(k, carry):
            @pl.when(k > 0)
            def _wg():
                wait_gather(tch)

            emit_groups(pl.multiple_of(t0 + k * tch, 8), tch, k == 0, tch)

            @plsc.parallel_loop(0, glen - tch)
            def copy_down(r):
                for bb in range(B):
                    for half in range(2):
                        vals = [
                            buf_in[bb, tch + r, pl.ds(384 * half + 16 * w, 16)]
                            for w in range(24)
                        ]
                        for w in range(24):
                            buf_in[bb, r, pl.ds(384 * half + 16 * w, 16)] = (
                                vals[w]
                            )

            pre = pl.multiple_of(t0 + k * tch + glen, 8)
            short = jnp.logical_and(k == 1, wid == nsc - 1)

            @pl.when(jnp.logical_not(short))
            def _pre_full():
                pltpu.async_copy(
                    x_hbm.at[:, pl.ds(pre, tch), :],
                    buf_in.at[:, pl.ds(glen - tch, tch), :],
                    gsem,
                )

            @pl.when(short)
            def _pre_short():
                pltpu.async_copy(
                    x_hbm.at[:, pl.ds(pre, 16), :],
                    buf_in.at[:, pl.ds(glen - tch, 16), :],
                    gsem,
                )

            return carry

        lax.fori_loop(0, 2, chunk_body, 0)

        @pl.when(wid < nsc - 1)
        def _wg2_full():
            wait_gather(tch)

        @pl.when(wid == nsc - 1)
        def _wg2_short():
            wait_gather(16)

        emit_groups(pl.multiple_of(t0 + 2 * tch, 8), tail_l2, None, tch)
        wait_scatter(tail_l2)
        wait_scatter(tail_l2)

    out3 = run(x)
    return (
        out3.reshape(valid_t, nwin * (C // 128), B, 128)
        .transpose(2, 0, 1, 3)
        .reshape(B, valid_t, nwin * C)
    )
